# baseline (speedup 1.0000x reference)
import functools

import jax
import jax.numpy as jnp
from jax import lax
from jax.experimental import pallas as pl
from jax.experimental.pallas import tpu as pltpu
from jax.experimental.pallas import tpu_sc as plsc

_HIDDEN = 2048
_E = 64
_T = 16384
_BT = 1024
_NW = 32
_TPW = _T // _NW
_L = 16
_NG = _TPW // _L


def _tc_logits_body(x_ref, w_ref, o_ref):
    o_ref[...] = lax.dot_general(
        x_ref[...], w_ref[...],
        dimension_numbers=(((1,), (1,)), ((), ())),
        preferred_element_type=jnp.float32,
        precision=lax.Precision.DEFAULT,
    )


def _tc_logits(x, W):
    return pl.pallas_call(
        _tc_logits_body,
        grid=(_T // _BT,),
        in_specs=[
            pl.BlockSpec((_BT, _HIDDEN), lambda i: (i, 0)),
            pl.BlockSpec((_E, _HIDDEN), lambda i: (0, 0)),
        ],
        out_specs=pl.BlockSpec((_BT, _E), lambda i: (i, 0)),
        out_shape=jax.ShapeDtypeStruct((_T, _E), jnp.float32),
    )(x, W)


def _sc_route(logits):
    mesh = plsc.VectorSubcoreMesh(core_axis_name="c", subcore_axis_name="s")

    @functools.partial(
        pl.kernel,
        mesh=mesh,
        compiler_params=pltpu.CompilerParams(needs_layout_passes=False),
        out_type=[
            jax.ShapeDtypeStruct((_T,), jnp.float32),
            jax.ShapeDtypeStruct((_T,), jnp.float32),
            jax.ShapeDtypeStruct((_T,), jnp.int32),
            jax.ShapeDtypeStruct((_T,), jnp.int32),
        ],
        scratch_types=[
            pltpu.VMEM((_TPW * _E,), jnp.float32),
            pltpu.VMEM((_TPW,), jnp.float32),
            pltpu.VMEM((_TPW,), jnp.float32),
            pltpu.VMEM((_TPW,), jnp.int32),
            pltpu.VMEM((_TPW,), jnp.int32),
        ],
    )
    def k(logits_hbm, w1_hbm, w2_hbm, e1_hbm, e2_hbm, lv, w1v, w2v, e1v, e2v):
        wid = lax.axis_index("s") * 2 + lax.axis_index("c")
        base = wid * _TPW
        pltpu.sync_copy(logits_hbm.at[pl.ds(base * _E, _TPW * _E)], lv)

        iota = lax.iota(jnp.int32, _L)
        zeros_i = jnp.zeros((_L,), jnp.int32)
        neg = jnp.full((_L,), -1e30, jnp.float32)

        for g in range(_NG):
            flat0 = (g * _L + iota) * _E

            def estep(e, carry):
                m1, i1, m2, i2 = carry
                cols = e + zeros_i
                v = plsc.load_gather(lv, [flat0 + cols])
                gt1 = v > m1
                gt2 = v > m2
                i2 = jnp.where(gt1, i1, jnp.where(gt2, cols, i2))
                m2 = jnp.where(gt1, m1, jnp.where(gt2, v, m2))
                i1 = jnp.where(gt1, cols, i1)
                m1 = jnp.where(gt1, v, m1)
                return m1, i1, m2, i2

            m1, i1, m2, i2 = lax.fori_loop(
                0, _E, estep, (neg, zeros_i, neg, zeros_i))

            def sstep(e, s):
                v = plsc.load_gather(lv, [flat0 + e + zeros_i])
                return s + jnp.exp(v - m1)

            s = lax.fori_loop(0, _E, sstep, jnp.zeros((_L,), jnp.float32))

            p2 = jnp.exp(m2 - m1)
            w1 = 1.0 / (1.0 + p2 + 1e-6 * s)
            w2 = p2 * w1

            w1v[pl.ds(g * _L, _L)] = w1
            w2v[pl.ds(g * _L, _L)] = w2
            e1v[pl.ds(g * _L, _L)] = i1
            e2v[pl.ds(g * _L, _L)] = i2

        pltpu.sync_copy(w1v, w1_hbm.at[pl.ds(base, _TPW)])
        pltpu.sync_copy(w2v, w2_hbm.at[pl.ds(base, _TPW)])
        pltpu.sync_copy(e1v, e1_hbm.at[pl.ds(base, _TPW)])
        pltpu.sync_copy(e2v, e2_hbm.at[pl.ds(base, _TPW)])

    return k(logits)


def kernel(hidden_states, W):
    b, s, h = hidden_states.shape
    x = hidden_states.reshape(b * s, h)
    logits = _tc_logits(x, W)
    w1, w2, e1, e2 = _sc_route(logits.reshape(_T * _E))
    routing_weights = jnp.stack([w1, w2], axis=-1).reshape(b, s, 2)
    selected_experts = jnp.stack([e1, e2], axis=-1).reshape(b, s, 2)
    router_logits = logits.reshape(b, s, _E)
    return routing_weights, selected_experts, router_logits


# speedup vs baseline: 1.2273x; 1.2273x over previous
import functools

import jax
import jax.numpy as jnp
from jax import lax
from jax.experimental import pallas as pl
from jax.experimental.pallas import tpu as pltpu
from jax.experimental.pallas import tpu_sc as plsc

_HIDDEN = 2048
_E = 64
_T = 16384
_BT = 1024
_NW = 32
_TPW = _T // _NW
_L = 16
_NG = _TPW // _L
_NC = 4


def _tc_logits_body(x_ref, w_ref, o_ref, s_ref):
    l = lax.dot_general(
        x_ref[...], w_ref[...],
        dimension_numbers=(((1,), (1,)), ((), ())),
        preferred_element_type=jnp.float32,
        precision=lax.Precision.DEFAULT,
    )
    o_ref[...] = l
    m = jnp.max(l, axis=1, keepdims=True)
    s_ref[...] = jnp.sum(jnp.exp(l - m), axis=1)


def _tc_logits(x, W):
    return pl.pallas_call(
        _tc_logits_body,
        grid=(_T // _BT,),
        in_specs=[
            pl.BlockSpec((_BT, _HIDDEN), lambda i: (i, 0)),
            pl.BlockSpec((_E, _HIDDEN), lambda i: (0, 0)),
        ],
        out_specs=[
            pl.BlockSpec((_BT, _E), lambda i: (i, 0)),
            pl.BlockSpec((_BT,), lambda i: (i,)),
        ],
        out_shape=[
            jax.ShapeDtypeStruct((_T, _E), jnp.float32),
            jax.ShapeDtypeStruct((_T,), jnp.float32),
        ],
    )(x, W)


def _pick(ma, ia, mb, ib):
    take_a = (ma > mb) | ((ma == mb) & (ia < ib))
    return jnp.where(take_a, ma, mb), jnp.where(take_a, ia, ib)


def _merge_top2(a, b):
    m1a, i1a, m2a, i2a = a
    m1b, i1b, m2b, i2b = b
    m1, i1 = _pick(m1a, i1a, m1b, i1b)
    a_won = (m1a > m1b) | ((m1a == m1b) & (i1a < i1b))
    c1m = jnp.where(a_won, m2a, m1a)
    c1i = jnp.where(a_won, i2a, i1a)
    c2m = jnp.where(a_won, m1b, m2b)
    c2i = jnp.where(a_won, i1b, i2b)
    m2, i2 = _pick(c1m, c1i, c2m, c2i)
    return m1, i1, m2, i2


def _sc_route(logits_flat, s_all):
    mesh = plsc.VectorSubcoreMesh(core_axis_name="c", subcore_axis_name="s")

    @functools.partial(
        pl.kernel,
        mesh=mesh,
        compiler_params=pltpu.CompilerParams(needs_layout_passes=False),
        out_type=[
            jax.ShapeDtypeStruct((_T,), jnp.float32),
            jax.ShapeDtypeStruct((_T,), jnp.float32),
            jax.ShapeDtypeStruct((_T,), jnp.int32),
            jax.ShapeDtypeStruct((_T,), jnp.int32),
        ],
        scratch_types=[
            pltpu.VMEM((_TPW * _E,), jnp.float32),
            pltpu.VMEM((_TPW,), jnp.float32),
            pltpu.VMEM((_TPW,), jnp.float32),
            pltpu.VMEM((_TPW,), jnp.float32),
            pltpu.VMEM((_TPW,), jnp.int32),
            pltpu.VMEM((_TPW,), jnp.int32),
        ],
    )
    def k(logits_hbm, s_hbm, w1_hbm, w2_hbm, e1_hbm, e2_hbm,
          lv, sv, w1v, w2v, e1v, e2v):
        wid = lax.axis_index("s") * 2 + lax.axis_index("c")
        base = wid * _TPW
        pltpu.sync_copy(logits_hbm.at[pl.ds(base * _E, _TPW * _E)], lv)
        pltpu.sync_copy(s_hbm.at[pl.ds(base, _TPW)], sv)

        iota = lax.iota(jnp.int32, _L)
        neg = jnp.full((_L,), -1e30, jnp.float32)

        def group_body(g, _):
            flat0 = (g * _L + iota) * _E
            chains = []
            for c in range(_NC):
                e0 = c * (_E // _NC)
                m1 = plsc.load_gather(lv, [flat0 + e0])
                i1 = jnp.full((_L,), e0, jnp.int32)
                m2 = neg
                i2 = i1
                for e in range(e0 + 1, e0 + _E // _NC):
                    v = plsc.load_gather(lv, [flat0 + e])
                    es = jnp.full((_L,), e, jnp.int32)
                    gt1 = v > m1
                    gt2 = v > m2
                    i2 = jnp.where(gt1, i1, jnp.where(gt2, es, i2))
                    m2 = jnp.where(gt1, m1, jnp.where(gt2, v, m2))
                    i1 = jnp.where(gt1, es, i1)
                    m1 = jnp.where(gt1, v, m1)
                chains.append((m1, i1, m2, i2))
            t01 = _merge_top2(chains[0], chains[1])
            t23 = _merge_top2(chains[2], chains[3])
            m1, i1, m2, i2 = _merge_top2(t01, t23)

            s = sv[pl.ds(g * _L, _L)]
            p2 = jnp.exp(m2 - m1)
            w1 = 1.0 / (1.0 + p2 + 1e-6 * s)
            w2 = p2 * w1

            w1v[pl.ds(g * _L, _L)] = w1
            w2v[pl.ds(g * _L, _L)] = w2
            e1v[pl.ds(g * _L, _L)] = i1
            e2v[pl.ds(g * _L, _L)] = i2
            return 0

        lax.fori_loop(0, _NG, group_body, 0)

        pltpu.sync_copy(w1v, w1_hbm.at[pl.ds(base, _TPW)])
        pltpu.sync_copy(w2v, w2_hbm.at[pl.ds(base, _TPW)])
        pltpu.sync_copy(e1v, e1_hbm.at[pl.ds(base, _TPW)])
        pltpu.sync_copy(e2v, e2_hbm.at[pl.ds(base, _TPW)])

    return k(logits_flat, s_all)


def kernel(hidden_states, W):
    b, s, h = hidden_states.shape
    x = hidden_states.reshape(b * s, h)
    logits, ssum = _tc_logits(x, W)
    w1, w2, e1, e2 = _sc_route(logits.reshape(_T * _E), ssum)
    routing_weights = jnp.stack([w1, w2], axis=-1).reshape(b, s, 2)
    selected_experts = jnp.stack([e1, e2], axis=-1).reshape(b, s, 2)
    router_logits = logits.reshape(b, s, _E)
    return routing_weights, selected_experts, router_logits


# speedup vs baseline: 1.3829x; 1.1268x over previous
import functools

import jax
import jax.numpy as jnp
from jax import lax
from jax.experimental import pallas as pl
from jax.experimental.pallas import tpu as pltpu
from jax.experimental.pallas import tpu_sc as plsc

_HIDDEN = 2048
_E = 64
_T = 16384
_BT = 1024
_NW = 32
_TPW = _T // _NW
_L = 16
_NG = _TPW // _L
_NC = 4


def _tc_logits_body(x_ref, w_ref, o_ref, s_ref):
    l = lax.dot_general(
        x_ref[...], w_ref[...],
        dimension_numbers=(((1,), (1,)), ((), ())),
        preferred_element_type=jnp.float32,
        precision=lax.Precision.DEFAULT,
    )
    o_ref[...] = l
    ones = jnp.ones((_E, 128), jnp.float32)
    sfull = lax.dot_general(
        jnp.exp(l), ones,
        dimension_numbers=(((1,), (0,)), ((), ())),
        preferred_element_type=jnp.float32,
        precision=lax.Precision.DEFAULT,
    )
    s_ref[...] = sfull[:, 0]


def _tc_logits(x, W):
    return pl.pallas_call(
        _tc_logits_body,
        grid=(_T // _BT,),
        in_specs=[
            pl.BlockSpec((_BT, _HIDDEN), lambda i: (i, 0)),
            pl.BlockSpec((_E, _HIDDEN), lambda i: (0, 0)),
        ],
        out_specs=[
            pl.BlockSpec((_BT, _E), lambda i: (i, 0)),
            pl.BlockSpec((_BT,), lambda i: (i,)),
        ],
        out_shape=[
            jax.ShapeDtypeStruct((_T, _E), jnp.float32),
            jax.ShapeDtypeStruct((_T,), jnp.float32),
        ],
    )(x, W)


def _pick(ma, ia, mb, ib):
    take_a = (ma > mb) | ((ma == mb) & (ia < ib))
    return jnp.where(take_a, ma, mb), jnp.where(take_a, ia, ib)


def _merge_top2(a, b):
    m1a, i1a, m2a, i2a = a
    m1b, i1b, m2b, i2b = b
    m1, i1 = _pick(m1a, i1a, m1b, i1b)
    a_won = (m1a > m1b) | ((m1a == m1b) & (i1a < i1b))
    c1m = jnp.where(a_won, m2a, m1a)
    c1i = jnp.where(a_won, i2a, i1a)
    c2m = jnp.where(a_won, m1b, m2b)
    c2i = jnp.where(a_won, i1b, i2b)
    m2, i2 = _pick(c1m, c1i, c2m, c2i)
    return m1, i1, m2, i2


def _sc_route(logits2d, s_all):
    mesh = plsc.VectorSubcoreMesh(core_axis_name="c", subcore_axis_name="s")

    @functools.partial(
        pl.kernel,
        mesh=mesh,
        compiler_params=pltpu.CompilerParams(needs_layout_passes=False),
        out_type=[
            jax.ShapeDtypeStruct((_T,), jnp.float32),
            jax.ShapeDtypeStruct((_T,), jnp.float32),
            jax.ShapeDtypeStruct((_T,), jnp.int32),
            jax.ShapeDtypeStruct((_T,), jnp.int32),
        ],
        scratch_types=[
            pltpu.VMEM((_TPW, _E), jnp.float32),
            pltpu.VMEM((_TPW,), jnp.float32),
            pltpu.VMEM((_TPW,), jnp.float32),
            pltpu.VMEM((_TPW,), jnp.float32),
            pltpu.VMEM((_TPW,), jnp.int32),
            pltpu.VMEM((_TPW,), jnp.int32),
        ],
    )
    def k(logits_hbm, s_hbm, w1_hbm, w2_hbm, e1_hbm, e2_hbm,
          lv, sv, w1v, w2v, e1v, e2v):
        wid = lax.axis_index("s") * 2 + lax.axis_index("c")
        base = wid * _TPW
        pltpu.sync_copy(logits_hbm.at[pl.ds(base, _TPW)], lv)
        pltpu.sync_copy(s_hbm.at[pl.ds(base, _TPW)], sv)

        iota = lax.iota(jnp.int32, _L)
        neg = jnp.full((_L,), -1e30, jnp.float32)

        def group_body(g, _):
            rows = g * _L + iota
            chains = []
            for c in range(_NC):
                e0 = c * (_E // _NC)
                m1 = plsc.load_gather(lv, [rows, jnp.full((_L,), e0, jnp.int32)])
                i1 = jnp.full((_L,), e0, jnp.int32)
                m2 = neg
                i2 = i1
                for e in range(e0 + 1, e0 + _E // _NC):
                    v = plsc.load_gather(lv, [rows, jnp.full((_L,), e, jnp.int32)])
                    es = jnp.full((_L,), e, jnp.int32)
                    gt1 = v > m1
                    gt2 = v > m2
                    i2 = jnp.where(gt1, i1, jnp.where(gt2, es, i2))
                    m2 = jnp.where(gt1, m1, jnp.where(gt2, v, m2))
                    i1 = jnp.where(gt1, es, i1)
                    m1 = jnp.where(gt1, v, m1)
                chains.append((m1, i1, m2, i2))
            t01 = _merge_top2(chains[0], chains[1])
            t23 = _merge_top2(chains[2], chains[3])
            m1, i1, m2, i2 = _merge_top2(t01, t23)

            s = sv[pl.ds(g * _L, _L)] * jnp.exp(-m1)
            p2 = jnp.exp(m2 - m1)
            w1 = 1.0 / (1.0 + p2 + 1e-6 * s)
            w2 = p2 * w1

            w1v[pl.ds(g * _L, _L)] = w1
            w2v[pl.ds(g * _L, _L)] = w2
            e1v[pl.ds(g * _L, _L)] = i1
            e2v[pl.ds(g * _L, _L)] = i2
            return 0

        lax.fori_loop(0, _NG, group_body, 0)

        pltpu.sync_copy(w1v, w1_hbm.at[pl.ds(base, _TPW)])
        pltpu.sync_copy(w2v, w2_hbm.at[pl.ds(base, _TPW)])
        pltpu.sync_copy(e1v, e1_hbm.at[pl.ds(base, _TPW)])
        pltpu.sync_copy(e2v, e2_hbm.at[pl.ds(base, _TPW)])

    return k(logits2d, s_all)


def kernel(hidden_states, W):
    b, s, h = hidden_states.shape
    x = hidden_states.reshape(b * s, h)
    logits, ssum = _tc_logits(x, W)
    w1, w2, e1, e2 = _sc_route(logits, ssum)
    routing_weights = jnp.stack([w1, w2], axis=-1).reshape(b, s, 2)
    selected_experts = jnp.stack([e1, e2], axis=-1).reshape(b, s, 2)
    router_logits = logits.reshape(b, s, _E)
    return routing_weights, selected_experts, router_logits
